# initial kernel scaffold (unmeasured)
import jax
import jax.numpy as jnp
from jax import lax
from jax.experimental import pallas as pl
from jax.experimental.pallas import tpu as pltpu


def kernel(
    x,
):
    def body(*refs):
        pass

    out_shape = jax.ShapeDtypeStruct(..., jnp.float32)
    return pl.pallas_call(body, out_shape=out_shape)(...)


# baseline (device time: 19764 ns/iter reference)
import jax
import jax.numpy as jnp
from jax import lax
from jax.experimental import pallas as pl
from jax.experimental.pallas import tpu as pltpu


def kernel(x):
    m, n = x.shape

    def body(x_ref, out_ref, comm_ref, send_sems, recv_sems):
        my_x = lax.axis_index("x")
        my_y = lax.axis_index("y")
        y_nbr = (my_x, 1 - my_y)
        x_nbr = (1 - my_x, my_y)

        barrier_sem = pltpu.get_barrier_semaphore()
        for nbr in (y_nbr, x_nbr):
            pl.semaphore_signal(
                barrier_sem, inc=1,
                device_id=nbr, device_id_type=pl.DeviceIdType.MESH,
            )
        pl.semaphore_wait(barrier_sem, 2)

        comm_ref[0, :, :] = x_ref[:, :].astype(jnp.bfloat16)
        rdma1 = pltpu.make_async_remote_copy(
            src_ref=comm_ref.at[0],
            dst_ref=comm_ref.at[1],
            send_sem=send_sems.at[0],
            recv_sem=recv_sems.at[0],
            device_id=y_nbr,
            device_id_type=pl.DeviceIdType.MESH,
        )
        rdma1.start()
        rdma1.wait()
        acc = x_ref[:, :] + comm_ref[1, :, :].astype(jnp.float32)

        comm_ref[0, :, :] = acc.astype(jnp.bfloat16)
        rdma2 = pltpu.make_async_remote_copy(
            src_ref=comm_ref.at[0],
            dst_ref=comm_ref.at[2],
            send_sem=send_sems.at[1],
            recv_sem=recv_sems.at[1],
            device_id=x_nbr,
            device_id_type=pl.DeviceIdType.MESH,
        )
        rdma2.start()
        rdma2.wait()
        out_ref[:, :] = acc + comm_ref[2, :, :].astype(jnp.float32)

    return pl.pallas_call(
        body,
        out_shape=jax.ShapeDtypeStruct((m, n), jnp.float32),
        in_specs=[pl.BlockSpec(memory_space=pltpu.VMEM)],
        out_specs=pl.BlockSpec(memory_space=pltpu.VMEM),
        scratch_shapes=[
            pltpu.VMEM((3, m, n), jnp.bfloat16),
            pltpu.SemaphoreType.DMA((2,)),
            pltpu.SemaphoreType.DMA((2,)),
        ],
        compiler_params=pltpu.CompilerParams(collective_id=0),
    )(x)


# device time: 14195 ns/iter; 1.3923x vs baseline; 1.3923x over previous
import jax
import jax.numpy as jnp
from jax import lax
from jax.experimental import pallas as pl
from jax.experimental.pallas import tpu as pltpu


def kernel(x):
    m, n = x.shape
    h = m // 2

    def body(x_ref, out_ref, comm_ref, send_sems, recv_sems):
        my_x = lax.axis_index("x")
        my_y = lax.axis_index("y")
        y_nbr = (my_x, 1 - my_y)
        x_nbr = (1 - my_x, my_y)

        barrier_sem = pltpu.get_barrier_semaphore()
        for nbr in (y_nbr, x_nbr):
            pl.semaphore_signal(
                barrier_sem, inc=1,
                device_id=nbr, device_id_type=pl.DeviceIdType.MESH,
            )
        pl.semaphore_wait(barrier_sem, 2)

        comm_ref[0, :, :] = x_ref[0:h, :].astype(jnp.bfloat16)
        comm_ref[1, :, :] = x_ref[h:m, :].astype(jnp.bfloat16)

        r1 = pltpu.make_async_remote_copy(
            src_ref=comm_ref.at[0], dst_ref=comm_ref.at[2],
            send_sem=send_sems.at[0], recv_sem=recv_sems.at[0],
            device_id=y_nbr, device_id_type=pl.DeviceIdType.MESH,
        )
        r2 = pltpu.make_async_remote_copy(
            src_ref=comm_ref.at[1], dst_ref=comm_ref.at[3],
            send_sem=send_sems.at[1], recv_sem=recv_sems.at[1],
            device_id=x_nbr, device_id_type=pl.DeviceIdType.MESH,
        )
        r1.start()
        r2.start()

        r1.wait_recv()
        acc_top = x_ref[0:h, :] + comm_ref[2, :, :].astype(jnp.float32)
        comm_ref[4, :, :] = acc_top.astype(jnp.bfloat16)
        r3 = pltpu.make_async_remote_copy(
            src_ref=comm_ref.at[4], dst_ref=comm_ref.at[6],
            send_sem=send_sems.at[2], recv_sem=recv_sems.at[2],
            device_id=x_nbr, device_id_type=pl.DeviceIdType.MESH,
        )
        r3.start()

        r2.wait_recv()
        acc_bot = x_ref[h:m, :] + comm_ref[3, :, :].astype(jnp.float32)
        comm_ref[5, :, :] = acc_bot.astype(jnp.bfloat16)
        r4 = pltpu.make_async_remote_copy(
            src_ref=comm_ref.at[5], dst_ref=comm_ref.at[7],
            send_sem=send_sems.at[3], recv_sem=recv_sems.at[3],
            device_id=y_nbr, device_id_type=pl.DeviceIdType.MESH,
        )
        r4.start()

        r3.wait_recv()
        out_ref[0:h, :] = acc_top + comm_ref[6, :, :].astype(jnp.float32)
        r4.wait_recv()
        out_ref[h:m, :] = acc_bot + comm_ref[7, :, :].astype(jnp.float32)

        r1.wait_send()
        r2.wait_send()
        r3.wait_send()
        r4.wait_send()

    return pl.pallas_call(
        body,
        out_shape=jax.ShapeDtypeStruct((m, n), jnp.float32),
        in_specs=[pl.BlockSpec(memory_space=pltpu.VMEM)],
        out_specs=pl.BlockSpec(memory_space=pltpu.VMEM),
        scratch_shapes=[
            pltpu.VMEM((8, h, n), jnp.bfloat16),
            pltpu.SemaphoreType.DMA((4,)),
            pltpu.SemaphoreType.DMA((4,)),
        ],
        compiler_params=pltpu.CompilerParams(collective_id=0),
    )(x)


# device time: 12940 ns/iter; 1.5274x vs baseline; 1.0970x over previous
import jax
import jax.numpy as jnp
from jax import lax
from jax.experimental import pallas as pl
from jax.experimental.pallas import tpu as pltpu

NQ = 4


def kernel(x):
    m, n = x.shape
    h = m // NQ

    def body(x_ref, out_ref, comm_ref, send_sems, recv_sems):
        my_x = lax.axis_index("x")
        my_y = lax.axis_index("y")
        y_nbr = (my_x, 1 - my_y)
        x_nbr = (1 - my_x, my_y)

        barrier_sem = pltpu.get_barrier_semaphore()
        for nbr in (y_nbr, x_nbr):
            pl.semaphore_signal(
                barrier_sem, inc=1,
                device_id=nbr, device_id_type=pl.DeviceIdType.MESH,
            )
        pl.semaphore_wait(barrier_sem, 2)

        qs = [
            (0, y_nbr, x_nbr),
            (2, x_nbr, y_nbr),
            (1, y_nbr, x_nbr),
            (3, x_nbr, y_nbr),
        ]

        p1 = {}
        for q, nbr1, _ in qs:
            comm_ref[q, :, :] = x_ref[q * h:(q + 1) * h, :].astype(jnp.bfloat16)
            r = pltpu.make_async_remote_copy(
                src_ref=comm_ref.at[q], dst_ref=comm_ref.at[4 + q],
                send_sem=send_sems.at[q], recv_sem=recv_sems.at[q],
                device_id=nbr1, device_id_type=pl.DeviceIdType.MESH,
            )
            r.start()
            p1[q] = r

        p2 = {}
        acc = {}
        for q, _, nbr2 in qs:
            p1[q].wait_recv()
            acc[q] = (
                x_ref[q * h:(q + 1) * h, :]
                + comm_ref[4 + q, :, :].astype(jnp.float32)
            )
            comm_ref[8 + q, :, :] = acc[q].astype(jnp.bfloat16)
            r = pltpu.make_async_remote_copy(
                src_ref=comm_ref.at[8 + q], dst_ref=comm_ref.at[12 + q],
                send_sem=send_sems.at[NQ + q], recv_sem=recv_sems.at[NQ + q],
                device_id=nbr2, device_id_type=pl.DeviceIdType.MESH,
            )
            r.start()
            p2[q] = r

        for q, _, _ in qs:
            p2[q].wait_recv()
            out_ref[q * h:(q + 1) * h, :] = (
                acc[q] + comm_ref[12 + q, :, :].astype(jnp.float32)
            )

        for q in range(NQ):
            p1[q].wait_send()
            p2[q].wait_send()

    return pl.pallas_call(
        body,
        out_shape=jax.ShapeDtypeStruct((m, n), jnp.float32),
        in_specs=[pl.BlockSpec(memory_space=pltpu.VMEM)],
        out_specs=pl.BlockSpec(memory_space=pltpu.VMEM),
        scratch_shapes=[
            pltpu.VMEM((4 * NQ, h, n), jnp.bfloat16),
            pltpu.SemaphoreType.DMA((2 * NQ,)),
            pltpu.SemaphoreType.DMA((2 * NQ,)),
        ],
        compiler_params=pltpu.CompilerParams(collective_id=0),
    )(x)


# device time: 12898 ns/iter; 1.5323x vs baseline; 1.0033x over previous
import jax
import jax.numpy as jnp
from jax import lax
from jax.experimental import pallas as pl
from jax.experimental.pallas import tpu as pltpu

NQ = 4


def kernel(x):
    m, n = x.shape
    h = m // NQ

    def body(x_ref, out_ref, comm_ref, send_sems, recv_sems):
        my_x = lax.axis_index("x")
        my_y = lax.axis_index("y")
        y_nbr = (my_x, 1 - my_y)
        x_nbr = (1 - my_x, my_y)

        barrier_sem = pltpu.get_barrier_semaphore()
        for nbr in (y_nbr, x_nbr):
            pl.semaphore_signal(
                barrier_sem, inc=1,
                device_id=nbr, device_id_type=pl.DeviceIdType.MESH,
            )
        pl.semaphore_wait(barrier_sem, 2)

        qs = [
            (0, y_nbr, x_nbr),
            (2, x_nbr, y_nbr),
            (1, y_nbr, x_nbr),
            (3, x_nbr, y_nbr),
        ]

        p1 = {}
        for q, nbr1, _ in qs:
            comm_ref[q, :, :] = x_ref[q * h:(q + 1) * h, :].astype(jnp.bfloat16)
            r = pltpu.make_async_remote_copy(
                src_ref=comm_ref.at[q], dst_ref=comm_ref.at[4 + q],
                send_sem=send_sems.at[q], recv_sem=recv_sems.at[q],
                device_id=nbr1, device_id_type=pl.DeviceIdType.MESH,
            )
            r.start()
            p1[q] = r

        p2 = {}
        for q, _, nbr2 in qs:
            p1[q].wait_recv()
            comm_ref[8 + q, :, :] = comm_ref[q, :, :] + comm_ref[4 + q, :, :]
            r = pltpu.make_async_remote_copy(
                src_ref=comm_ref.at[8 + q], dst_ref=comm_ref.at[12 + q],
                send_sem=send_sems.at[NQ + q], recv_sem=recv_sems.at[NQ + q],
                device_id=nbr2, device_id_type=pl.DeviceIdType.MESH,
            )
            r.start()
            p2[q] = r

        for q, _, _ in qs:
            p2[q].wait_recv()
            out_ref[q * h:(q + 1) * h, :] = (
                comm_ref[8 + q, :, :] + comm_ref[12 + q, :, :]
            ).astype(jnp.float32)

        for q in range(NQ):
            p1[q].wait_send()
            p2[q].wait_send()

    return pl.pallas_call(
        body,
        out_shape=jax.ShapeDtypeStruct((m, n), jnp.float32),
        in_specs=[pl.BlockSpec(memory_space=pltpu.VMEM)],
        out_specs=pl.BlockSpec(memory_space=pltpu.VMEM),
        scratch_shapes=[
            pltpu.VMEM((4 * NQ, h, n), jnp.bfloat16),
            pltpu.SemaphoreType.DMA((2 * NQ,)),
            pltpu.SemaphoreType.DMA((2 * NQ,)),
        ],
        compiler_params=pltpu.CompilerParams(collective_id=0),
    )(x)
